# baseline (speedup 1.0000x reference)
import functools

import jax
import jax.numpy as jnp
from jax import lax
from jax.experimental import pallas as pl
from jax.experimental.pallas import tpu as pltpu
from jax.experimental.pallas import tpu_sc as plsc

HIDDEN = 1024
D_FF = 1024
N_EXP = 8
N_LOGITS = 16
TOKENS = 2048
TT = 256
NA = 2 * TOKENS
BT = 256
NTILES = 24
P_ALLOC = NTILES * BT
DUMMY = P_ALLOC - 1
NW = 16
APW = NA // NW


def _router_body(x_ref, wr_ref, br_ref, cb_ref, i1_ref, i2_ref, w1_ref,
                 w2_ref, zwx_ref):
    x = x_ref[...]
    logits = jax.lax.dot_general(
        x, wr_ref[...], (((1,), (0,)), ((), ())),
        preferred_element_type=jnp.float32) + br_ref[...]
    scores = jax.nn.sigmoid(logits)
    biased = scores + cb_ref[...]
    col = jax.lax.broadcasted_iota(jnp.int32, biased.shape, 1)
    m1 = jnp.max(biased, axis=1, keepdims=True)
    i1 = jnp.min(jnp.where(biased == m1, col, N_LOGITS), axis=1, keepdims=True)
    sel1 = col == i1
    w1 = jnp.sum(jnp.where(sel1, scores, 0.0), axis=1, keepdims=True)
    b2 = jnp.where(sel1, -jnp.inf, biased)
    m2 = jnp.max(b2, axis=1, keepdims=True)
    i2 = jnp.min(jnp.where(b2 == m2, col, N_LOGITS), axis=1, keepdims=True)
    sel2 = col == i2
    w2 = jnp.sum(jnp.where(sel2, scores, 0.0), axis=1, keepdims=True)

    i1_ref[...] = i1
    i2_ref[...] = i2
    w1_ref[...] = w1
    w2_ref[...] = w2
    zw = (jnp.where(i1 >= N_EXP, w1, 0.0) + jnp.where(i2 >= N_EXP, w2, 0.0))
    zwx_ref[...] = zw * x


def _i16(c):
    return jnp.full((16,), c, jnp.int32)


def _dyn_gather16(y, idx):
    return jax.lax.gather(
        y, idx[:, None],
        jax.lax.GatherDimensionNumbers(
            offset_dims=(), collapsed_slice_dims=(0,), start_index_map=(0,)),
        (1,), mode=jax.lax.GatherScatterMode.PROMISE_IN_BOUNDS)


def _prefix16(y, lanes):
    z = jnp.zeros((16,), jnp.int32)
    for sh in (1, 2, 4, 8):
        idx = jnp.maximum(lanes - _i16(sh), z)
        g = _dyn_gather16(y, idx)
        y = y + jnp.where(lanes >= _i16(sh), g, z)
    return y


def _dispatch_body(eid_hbm, wgt_hbm, pos_hbm, tok_hbm, wgts_hbm, meta_hbm,
                   eid_loc, wgt_loc, lrank_loc, pos_loc, tokv_loc, wgtm_loc,
                   zb_loc, zbf_loc, cnt_vec_loc, cnt_loc, meta_loc,
                   sem, cnt_sh):
    cid = lax.axis_index("c")
    sid = lax.axis_index("s")
    lanes = lax.iota(jnp.int32, 16)
    zeros = jnp.zeros((16,), jnp.int32)

    @pl.when(cid == 0)
    def _():
        w = sid
        base_a = w * APW
        pltpu.sync_copy(eid_hbm.at[pl.ds(base_a, APW)], eid_loc)
        pltpu.sync_copy(wgt_hbm.at[pl.ds(base_a, APW)], wgt_loc)

        spw = P_ALLOC // NW

        def body_z(i, _):
            zb_loc[pl.ds(i * 16, 16)] = zeros
            zbf_loc[pl.ds(i * 16, 16)] = jnp.zeros((16,), jnp.float32)
            return 0

        lax.fori_loop(0, spw // 16, body_z, 0)
        pltpu.sync_copy(zb_loc, tok_hbm.at[pl.ds(w * spw, spw)])
        pltpu.sync_copy(zbf_loc, wgts_hbm.at[pl.ds(w * spw, spw)])

        def body_a(i, run):
            v = eid_loc[pl.ds(i * 16, 16)]
            lr = zeros
            new_run = []
            for ex in range(N_EXP):
                m = v == _i16(ex)
                mi = jnp.where(m, _i16(1), zeros)
                y = _prefix16(mi, lanes)
                lr = jnp.where(m, run[ex] + y - _i16(1), lr)
                new_run.append(run[ex] + _dyn_gather16(y, _i16(15)))
            lrank_loc[pl.ds(i * 16, 16)] = lr
            return tuple(new_run)

        run = lax.fori_loop(0, APW // 16, body_a,
                            tuple(zeros for _ in range(N_EXP)))
        cvec = zeros
        for ex in range(N_EXP):
            cvec = jnp.where(lanes == _i16(ex), run[ex], cvec)
        cnt_vec_loc[...] = cvec
        pltpu.sync_copy(cnt_vec_loc, cnt_sh.at[pl.ds(w * 16, 16)])
        plsc.subcore_barrier()

        pltpu.sync_copy(cnt_sh, cnt_loc)
        tot = zeros
        pre = zeros
        for wp in range(NW):
            cw = cnt_loc[pl.ds(wp * 16, 16)]
            tot = tot + cw
            sel = jnp.minimum(jnp.maximum(w - wp, 0), 1)
            pre = pre + cw * _i16(sel)
        tiles = zeros
        for k in range(NA // BT):
            tiles = tiles + jnp.where(tot > _i16(k * BT), _i16(1), zeros)
        padded = tiles * _i16(BT)
        incl = _prefix16(padded, lanes)
        exc = incl - padded
        base = exc + pre
        base_sp = [_dyn_gather16(base, _i16(ex)) for ex in range(N_EXP)]

        def body_b(i, _):
            v = eid_loc[pl.ds(i * 16, 16)]
            lr = lrank_loc[pl.ds(i * 16, 16)]
            wv = wgt_loc[pl.ds(i * 16, 16)]
            b = _i16(DUMMY)
            for ex in range(N_EXP):
                b = jnp.where(v == _i16(ex), base_sp[ex] + lr, b)
            pos_loc[pl.ds(i * 16, 16)] = b
            tokv_loc[pl.ds(i * 16, 16)] = (
                (lanes + _i16(base_a + i * 16)) & _i16(TOKENS - 1))
            routed = v < _i16(N_EXP)
            wgtm_loc[pl.ds(i * 16, 16)] = jnp.where(
                routed, wv, jnp.zeros((16,), jnp.float32))
            return 0

        lax.fori_loop(0, APW // 16, body_b, 0)
        pltpu.sync_copy(pos_loc, pos_hbm.at[pl.ds(base_a, APW)])
        cp1 = pltpu.async_copy(tokv_loc, tok_hbm.at[pos_loc], sem)
        cp2 = pltpu.async_copy(wgtm_loc, wgts_hbm.at[pos_loc], sem)
        cp1.wait()
        cp2.wait()

        @pl.when(w == 0)
        def _():
            n_used_v = _dyn_gather16(_prefix16(tiles, lanes), _i16(15))
            hix = exc + padded
            for j in range(2):
                jv = (lanes + _i16(j * 16)) * _i16(BT)
                ej = zeros
                for ex in range(N_EXP):
                    lo = _dyn_gather16(exc, _i16(ex))
                    hi = _dyn_gather16(hix, _i16(ex))
                    ej = jnp.where((jv >= lo) & (jv < hi), _i16(ex), ej)
                ej = jnp.where(jv == _i16(24 * BT), n_used_v, ej)
                meta_loc[pl.ds(j * 16, 16)] = ej
            pltpu.sync_copy(meta_loc, meta_hbm)


def _xgather_body(xi_hbm, tok_hbm, xs_hbm, idx_loc, rows_loc, sem):
    cid = lax.axis_index("c")
    sid = lax.axis_index("s")
    wid = sid * 2 + cid
    spw = P_ALLOC // 32
    base = wid * spw
    nch = 8
    ch = spw // nch
    pltpu.sync_copy(tok_hbm.at[pl.ds(base, spw)], idx_loc)
    cps = [pltpu.async_copy(xi_hbm.at[idx_loc.at[pl.ds(c * ch, ch)]],
                            rows_loc.at[pl.ds(c * ch, ch)], sem)
           for c in range(nch)]
    for cp in cps:
        cp.wait()
    pltpu.sync_copy(rows_loc, xs_hbm.at[pl.ds(base, spw)])


def _gmm_body(meta_ref, xs_ref, w1_ref, w2_ref, wg_ref, out_ref):
    i = pl.program_id(0)
    n_used = meta_ref[24]

    @pl.when(i < n_used)
    def _():
        h = jnp.dot(xs_ref[...], w1_ref[0], preferred_element_type=jnp.float32)
        gate = h[:, :D_FF]
        up = h[:, D_FF:]
        act = gate * jax.nn.sigmoid(gate) * up
        o = jnp.dot(act.astype(jnp.bfloat16), w2_ref[0],
                    preferred_element_type=jnp.float32)
        out_ref[...] = wg_ref[...] * o

    @pl.when(i == NTILES - 1)
    def _():
        out_ref[...] = jnp.zeros_like(out_ref)


def _combine_body(zwx_hbm, outs_hbm, pos_hbm, out_hbm,
                  idx0_loc, idx1_loc, b0, b1, b2, b3, b4, b5, b6, sem):
    cid = lax.axis_index("c")
    sid = lax.axis_index("s")
    wid = sid * 2 + cid
    tpw = TOKENS // 32
    CH = 16
    t0 = wid * tpw
    pltpu.sync_copy(pos_hbm.at[pl.ds(t0, tpw)], idx0_loc)
    pltpu.sync_copy(pos_hbm.at[pl.ds(TOKENS + t0, tpw)], idx1_loc)
    bufs = [b0, b1, b2, b3, b4, b5, b6]
    cps = []
    for c in range(3):
        cps.append(pltpu.async_copy(
            outs_hbm.at[idx0_loc.at[pl.ds(c * CH, CH)]], bufs[2 * c], sem))
        cps.append(pltpu.async_copy(
            outs_hbm.at[idx1_loc.at[pl.ds(c * CH, CH)]],
            bufs[2 * c + 1], sem))
    cps.append(pltpu.async_copy(
        outs_hbm.at[idx0_loc.at[pl.ds(3 * CH, CH)]], b6, sem))
    for cp in cps:
        cp.wait()

    def process(g0, g1, c):
        def row(i, _):
            def vec(j, _):
                sl = pl.ds(j * 16, 16)
                g0[i, sl] = g0[i, sl] + g1[i, sl]
                return 0
            return lax.fori_loop(0, HIDDEN // 16, vec, 0)

        lax.fori_loop(0, CH, row, 0)
        pltpu.sync_copy(zwx_hbm.at[pl.ds(t0 + c * CH, CH)], g1)
        lax.fori_loop(0, CH, row, 0)
        pltpu.sync_copy(g0, out_hbm.at[pl.ds(t0 + c * CH, CH)])

    process(b0, b1, 0)
    cp_last = pltpu.async_copy(
        outs_hbm.at[idx1_loc.at[pl.ds(3 * CH, CH)]], b0, sem)
    process(b2, b3, 1)
    process(b4, b5, 2)
    cp_last.wait()
    process(b6, b0, 3)


_sc_mesh = plsc.VectorSubcoreMesh(core_axis_name="c", subcore_axis_name="s")

_dispatch = pl.kernel(
    _dispatch_body, mesh=_sc_mesh,
    out_type=[
        jax.ShapeDtypeStruct((NA,), jnp.int32),
        jax.ShapeDtypeStruct((P_ALLOC,), jnp.int32),
        jax.ShapeDtypeStruct((P_ALLOC,), jnp.float32),
        jax.ShapeDtypeStruct((32,), jnp.int32),
    ],
    scratch_types=[
        pltpu.VMEM((APW,), jnp.int32),
        pltpu.VMEM((APW,), jnp.float32),
        pltpu.VMEM((APW,), jnp.int32),
        pltpu.VMEM((APW,), jnp.int32),
        pltpu.VMEM((APW,), jnp.int32),
        pltpu.VMEM((APW,), jnp.float32),
        pltpu.VMEM((P_ALLOC // NW,), jnp.int32),
        pltpu.VMEM((P_ALLOC // NW,), jnp.float32),
        pltpu.VMEM((16,), jnp.int32),
        pltpu.VMEM((16 * NW,), jnp.int32),
        pltpu.VMEM((32,), jnp.int32),
        pltpu.SemaphoreType.DMA,
        pltpu.VMEM_SHARED((16 * NW,), jnp.int32),
    ],
)

_xgather = pl.kernel(
    _xgather_body, mesh=_sc_mesh,
    out_type=[jax.ShapeDtypeStruct((P_ALLOC, HIDDEN // 2), jnp.int32)],
    scratch_types=[
        pltpu.VMEM((P_ALLOC // 32,), jnp.int32),
        pltpu.VMEM((P_ALLOC // 32, HIDDEN // 2), jnp.int32),
        pltpu.SemaphoreType.DMA,
    ],
)

_combine = pl.kernel(
    _combine_body, mesh=_sc_mesh,
    out_type=[jax.ShapeDtypeStruct((TOKENS, HIDDEN), jnp.float32)],
    scratch_types=(
        [pltpu.VMEM((TOKENS // 32,), jnp.int32)] * 2
        + [pltpu.VMEM((16, HIDDEN), jnp.float32)] * 7
        + [pltpu.SemaphoreType.DMA]
    ),
)


def kernel(hidden_states, Wr, br, correction_bias, W1, W2):
    T = hidden_states.shape[0]
    nt = T // TT
    br2 = br.reshape(1, N_LOGITS)
    cb2 = correction_bias.reshape(1, N_LOGITS)

    i1, i2, w1, w2, zwx = pl.pallas_call(
        _router_body,
        grid=(nt,),
        in_specs=[
            pl.BlockSpec((TT, HIDDEN), lambda t: (t, 0)),
            pl.BlockSpec((HIDDEN, N_LOGITS), lambda t: (0, 0)),
            pl.BlockSpec((1, N_LOGITS), lambda t: (0, 0)),
            pl.BlockSpec((1, N_LOGITS), lambda t: (0, 0)),
        ],
        out_specs=[
            pl.BlockSpec((TT, 1), lambda t: (t, 0)),
            pl.BlockSpec((TT, 1), lambda t: (t, 0)),
            pl.BlockSpec((TT, 1), lambda t: (t, 0)),
            pl.BlockSpec((TT, 1), lambda t: (t, 0)),
            pl.BlockSpec((TT, HIDDEN), lambda t: (t, 0)),
        ],
        out_shape=[
            jax.ShapeDtypeStruct((T, 1), jnp.int32),
            jax.ShapeDtypeStruct((T, 1), jnp.int32),
            jax.ShapeDtypeStruct((T, 1), jnp.float32),
            jax.ShapeDtypeStruct((T, 1), jnp.float32),
            jax.ShapeDtypeStruct((T, HIDDEN), jnp.float32),
        ],
    )(hidden_states, Wr, br2, cb2)

    eid = jnp.concatenate([i1, i2], axis=0).reshape(NA)
    wgt = jnp.concatenate([w1, w2], axis=0).reshape(NA)

    pos, tok, wgs, meta = _dispatch(eid, wgt)
    xb = hidden_states.astype(jnp.bfloat16)
    xi = jax.lax.bitcast_convert_type(
        xb.reshape(T, HIDDEN // 2, 2), jnp.int32)
    (xsi,) = _xgather(xi, tok)
    xs = jax.lax.bitcast_convert_type(xsi, jnp.bfloat16).reshape(
        P_ALLOC, HIDDEN)

    outs = pl.pallas_call(
        _gmm_body,
        grid_spec=pltpu.PrefetchScalarGridSpec(
            num_scalar_prefetch=1,
            grid=(NTILES,),
            in_specs=[
                pl.BlockSpec((BT, HIDDEN), lambda i, m: (i, 0)),
                pl.BlockSpec((1, HIDDEN, 2 * D_FF), lambda i, m: (m[i], 0, 0)),
                pl.BlockSpec((1, D_FF, HIDDEN), lambda i, m: (m[i], 0, 0)),
                pl.BlockSpec((BT, 1), lambda i, m: (i, 0)),
            ],
            out_specs=pl.BlockSpec((BT, HIDDEN), lambda i, m: (i, 0)),
        ),
        out_shape=jax.ShapeDtypeStruct((P_ALLOC, HIDDEN), jnp.float32),
    )(meta, xs, W1.astype(jnp.bfloat16), W2.astype(jnp.bfloat16),
      wgs.reshape(P_ALLOC, 1))

    (out,) = _combine(zwx, outs, pos)
    return out


# speedup vs baseline: 4.3210x; 4.3210x over previous
import functools

import jax
import jax.numpy as jnp
from jax.experimental import pallas as pl
from jax.experimental.pallas import tpu as pltpu

HIDDEN = 1024
D_FF = 1024
N_EXP = 8
N_LOGITS = 16
TOKENS = 2048
TT = 256


def _router_body(x_ref, wr_ref, br_ref, cb_ref, gates_ref, zw_ref):
    x = x_ref[...]
    logits = jax.lax.dot_general(
        x, wr_ref[...], (((1,), (0,)), ((), ())),
        preferred_element_type=jnp.float32) + br_ref[...]
    scores = jax.nn.sigmoid(logits)
    biased = scores + cb_ref[...]
    col = jax.lax.broadcasted_iota(jnp.int32, biased.shape, 1)
    m1 = jnp.max(biased, axis=1, keepdims=True)
    i1 = jnp.min(jnp.where(biased == m1, col, N_LOGITS), axis=1, keepdims=True)
    sel1 = col == i1
    w1 = jnp.sum(jnp.where(sel1, scores, 0.0), axis=1, keepdims=True)
    b2 = jnp.where(sel1, -jnp.inf, biased)
    m2 = jnp.max(b2, axis=1, keepdims=True)
    i2 = jnp.min(jnp.where(b2 == m2, col, N_LOGITS), axis=1, keepdims=True)
    sel2 = col == i2
    w2 = jnp.sum(jnp.where(sel2, scores, 0.0), axis=1, keepdims=True)

    ecol = jax.lax.broadcasted_iota(jnp.int32, (TT, N_EXP), 1)
    g1 = jnp.where((i1 == ecol) & (i1 < N_EXP), w1, 0.0)
    g2 = jnp.where((i2 == ecol) & (i2 < N_EXP), w2, 0.0)
    gates_ref[...] = g1 + g2
    zw_ref[...] = (jnp.where(i1 >= N_EXP, w1, 0.0)
                   + jnp.where(i2 >= N_EXP, w2, 0.0))


def _moe_body(gates_ref, zw_ref, x_ref, w1_ref, w2_ref, out_ref):
    e = pl.program_id(1)
    x = x_ref[...]
    h = jnp.dot(x.astype(jnp.bfloat16), w1_ref[e],
                preferred_element_type=jnp.float32)
    gate = h[:, :D_FF]
    up = h[:, D_FF:]
    act = gate * jax.nn.sigmoid(gate) * up
    o = jnp.dot(act.astype(jnp.bfloat16), w2_ref[e],
                preferred_element_type=jnp.float32)
    ecol = jax.lax.broadcasted_iota(jnp.int32, (TT, N_EXP), 1)
    g = jnp.sum(jnp.where(ecol == e, gates_ref[...], 0.0), axis=1,
                keepdims=True)
    contrib = g * o

    @pl.when(e == 0)
    def _():
        out_ref[...] = zw_ref[...] * x + contrib

    @pl.when(e > 0)
    def _():
        out_ref[...] += contrib


def kernel(hidden_states, Wr, br, correction_bias, W1, W2):
    T = hidden_states.shape[0]
    nt = T // TT
    br2 = br.reshape(1, N_LOGITS)
    cb2 = correction_bias.reshape(1, N_LOGITS)

    gates, zw = pl.pallas_call(
        _router_body,
        grid=(nt,),
        in_specs=[
            pl.BlockSpec((TT, HIDDEN), lambda t: (t, 0)),
            pl.BlockSpec((HIDDEN, N_LOGITS), lambda t: (0, 0)),
            pl.BlockSpec((1, N_LOGITS), lambda t: (0, 0)),
            pl.BlockSpec((1, N_LOGITS), lambda t: (0, 0)),
        ],
        out_specs=[
            pl.BlockSpec((TT, N_EXP), lambda t: (t, 0)),
            pl.BlockSpec((TT, 1), lambda t: (t, 0)),
        ],
        out_shape=[
            jax.ShapeDtypeStruct((T, N_EXP), jnp.float32),
            jax.ShapeDtypeStruct((T, 1), jnp.float32),
        ],
    )(hidden_states, Wr, br2, cb2)

    out = pl.pallas_call(
        _moe_body,
        grid=(nt, N_EXP),
        in_specs=[
            pl.BlockSpec((TT, N_EXP), lambda t, e: (t, 0)),
            pl.BlockSpec((TT, 1), lambda t, e: (t, 0)),
            pl.BlockSpec((TT, HIDDEN), lambda t, e: (t, 0)),
            pl.BlockSpec((N_EXP, HIDDEN, 2 * D_FF), lambda t, e: (0, 0, 0)),
            pl.BlockSpec((N_EXP, D_FF, HIDDEN), lambda t, e: (0, 0, 0)),
        ],
        out_specs=pl.BlockSpec((TT, HIDDEN), lambda t, e: (t, 0)),
        out_shape=jax.ShapeDtypeStruct((T, HIDDEN), jnp.float32),
    )(gates, zw, hidden_states, W1.astype(jnp.bfloat16),
      W2.astype(jnp.bfloat16))
    return out
